# baseline (speedup 1.0000x reference)
import functools

import jax
import jax.numpy as jnp
from jax import lax
from jax.experimental import pallas as pl
from jax.experimental.pallas import tpu as pltpu
from jax.experimental.pallas import tpu_sc as plsc

_E = 1048576
_N = 32768
_R = 4
_NW = 32
_EPW = _E // _NW
_CH = 2048
_NCH = _EPW // _CH
_NBUK = 16
_BKT = _N // _NBUK
_BT = 128
_STG = _BT + 16
_CAPB = _EPW + _BT
_DUMP = _R * _BKT
_ACC_ROWS = _DUMP + 256
_W = 128


def _bucket_body(src_hbm, dst_hbm, rel_hbm, brs_hbm, brk_hbm, cnt_hbm,
                 sbuf, dbuf, rbuf, stgs, stgk, cntv):
    c = lax.axis_index("c")
    s = lax.axis_index("s")
    wid = 2 * s + c
    ebase = wid * _EPW

    def chunk_body(ch, carry):
        base = pl.multiple_of(ebase + ch * _CH, _CH)
        pltpu.sync_copy(src_hbm.at[pl.ds(base, _CH)], sbuf)
        pltpu.sync_copy(dst_hbm.at[pl.ds(base, _CH)], dbuf)
        pltpu.sync_copy(rel_hbm.at[pl.ds(base, _CH)], rbuf)

        def vec_body(i, cr):
            fills, offs = cr[:_NBUK], cr[_NBUK:]
            sv = sbuf[pl.ds(i * 16, 16)]
            dv = dbuf[pl.ds(i * 16, 16)]
            rv = rbuf[pl.ds(i * 16, 16)]
            bkt = lax.shift_right_logical(dv, 11)
            key = jnp.bitwise_or(lax.shift_left(rv, 11),
                                 jnp.bitwise_and(dv, _BKT - 1))
            nf, no = [], []
            for b in range(_NBUK):
                fill, off = fills[b], offs[b]
                m = bkt == b
                cb = jnp.sum(m.astype(jnp.int32))
                plsc.store_compressed(stgs.at[pl.ds(b * _STG + fill, 16)], sv, mask=m)
                plsc.store_compressed(stgk.at[pl.ds(b * _STG + fill, 16)], key, mask=m)
                fill = fill + cb

                def flush(fl, of, b=b):
                    ro = pl.multiple_of((wid * _NBUK + b) * _CAPB + of * _BT, _BT)
                    pltpu.sync_copy(stgs.at[pl.ds(b * _STG, _BT)],
                                    brs_hbm.at[pl.ds(ro, _BT)])
                    pltpu.sync_copy(stgk.at[pl.ds(b * _STG, _BT)],
                                    brk_hbm.at[pl.ds(ro, _BT)])
                    los = stgs[pl.ds(b * _STG + _BT, 16)]
                    lok = stgk[pl.ds(b * _STG + _BT, 16)]
                    stgs[pl.ds(b * _STG, 16)] = los
                    stgk[pl.ds(b * _STG, 16)] = lok
                    return fl - _BT, of + 1

                fill, off = lax.cond(fill >= _BT, flush,
                                     lambda fl, of: (fl, of), fill, off)
                nf.append(fill)
                no.append(off)
            return tuple(nf) + tuple(no)

        return lax.fori_loop(0, _CH // 16, vec_body, carry)

    carry = (jnp.int32(0),) * (2 * _NBUK)
    carry = lax.fori_loop(0, _NCH, chunk_body, carry)
    fills, offs = carry[:_NBUK], carry[_NBUK:]
    dsrc = jnp.zeros((16,), jnp.int32)
    dkey = jnp.full((16,), _DUMP, jnp.int32)
    for b in range(_NBUK):
        fill, off = fills[b], offs[b]

        def final_flush(fl, of, b=b):
            npad = (_BT - fl + 15) // 16

            def pad_body(j, _):
                stgs[pl.ds(b * _STG + fl + j * 16, 16)] = dsrc
                stgk[pl.ds(b * _STG + fl + j * 16, 16)] = dkey
                return 0

            lax.fori_loop(0, npad, pad_body, 0)
            ro = pl.multiple_of((wid * _NBUK + b) * _CAPB + of * _BT, _BT)
            pltpu.sync_copy(stgs.at[pl.ds(b * _STG, _BT)],
                            brs_hbm.at[pl.ds(ro, _BT)])
            pltpu.sync_copy(stgk.at[pl.ds(b * _STG, _BT)],
                            brk_hbm.at[pl.ds(ro, _BT)])
            return of + 1

        off = lax.cond(fill > 0, final_flush, lambda fl, of: of, fill, off)
        offs = offs[:b] + (off,) + offs[b + 1:]
    lanes = lax.iota(jnp.int32, 16)
    cvec = jnp.zeros((16,), jnp.int32)
    for b in range(_NBUK):
        cvec = jnp.where(lanes == b, offs[b], cvec)
    cntv[...] = cvec
    co = pl.multiple_of(wid * 128, 128)
    pltpu.sync_copy(cntv, cnt_hbm.at[pl.ds(co, 16)])


def _bucket_call(src, dst, rel):
    mesh = plsc.VectorSubcoreMesh(core_axis_name="c", subcore_axis_name="s")
    f = pl.kernel(
        _bucket_body,
        out_type=[
            jax.ShapeDtypeStruct((_NW * _NBUK * _CAPB,), jnp.int32),
            jax.ShapeDtypeStruct((_NW * _NBUK * _CAPB,), jnp.int32),
            jax.ShapeDtypeStruct((_NW * 128,), jnp.int32),
        ],
        mesh=mesh,
        compiler_params=pltpu.CompilerParams(needs_layout_passes=False),
        scratch_types=[
            pltpu.VMEM((_CH,), jnp.int32),
            pltpu.VMEM((_CH,), jnp.int32),
            pltpu.VMEM((_CH,), jnp.int32),
            pltpu.VMEM((_NBUK * _STG,), jnp.int32),
            pltpu.VMEM((_NBUK * _STG,), jnp.int32),
            pltpu.VMEM((_NBUK,), jnp.int32),
        ],
    )
    return f(src, dst, rel)


_PRE = 20
_HB = 64


def _seg_body(hn_hbm, brs_hbm, brk_hbm, cntf_hbm, sout_hbm,
              acc, s0, s1, s2, s3, s4, s5, k0, k1, k2, k3, k4, k5,
              r0, r1, r2, r3, r4, r5, sfull, kfull,
              sp0, sp1, kp0, kp1, cntv, psem, gsem, ssem):
    c = lax.axis_index("c")
    s = lax.axis_index("s")
    sbufs = (s0, s1, s2, s3, s4, s5)
    kbufs = (k0, k1, k2, k3, k4, k5)
    rbufs = (r0, r1, r2, r3, r4, r5)
    st0 = 2 * s
    co = pl.multiple_of(st0 * 128, 128)
    pltpu.sync_copy(cntf_hbm.at[pl.ds(co, 256)], cntv.at[pl.ds(0, 256)])
    zv = jnp.zeros((16,), jnp.float32)

    def zrow(i, _):
        for j in range(_W // 16):
            r0[i, pl.ds(j * 16, 16)] = zv
        return 0

    def wait_gat(j):
        pltpu.make_async_copy(hn_hbm.at[sbufs[j]], rbufs[j], gsem.at[j]).wait()

    def wait_sca(j):
        pltpu.make_async_copy(rbufs[j], acc.at[kbufs[j]], ssem.at[j]).wait()

    def vcopy64(srcref, soff, dstref):
        for q in range(4):
            dstref[pl.ds(q * 16, 16)] = srcref[pl.ds(soff + q * 16, 16)]

    def bucket_body(bl, _):
        bucket = c * 8 + bl
        b0 = pl.multiple_of((st0 * _NBUK + bucket) * _CAPB, 128)
        b1 = pl.multiple_of(((st0 + 1) * _NBUK + bucket) * _CAPB, 128)
        pltpu.async_copy(brs_hbm.at[pl.ds(b0, _PRE * _BT)], sp0, psem)
        pltpu.async_copy(brk_hbm.at[pl.ds(b0, _PRE * _BT)], kp0, psem)
        pltpu.async_copy(brs_hbm.at[pl.ds(b1, _PRE * _BT)], sp1, psem)
        pltpu.async_copy(brk_hbm.at[pl.ds(b1, _PRE * _BT)], kp1, psem)
        lax.fori_loop(0, _HB, zrow, 0)
        row0 = s * (_ACC_ROWS // 16)
        for q in range(8):
            pltpu.sync_copy(r0, acc.at[pl.ds(row0 + q * _HB, _HB)])
        pltpu.sync_copy(r0.at[pl.ds(0, 16)], acc.at[pl.ds(row0 + 8 * _HB, 16)])
        for pb in (sp0, kp0, sp1, kp1):
            pltpu.make_async_copy(brs_hbm.at[pl.ds(0, _PRE * _BT)], pb, psem).wait()
        plsc.subcore_barrier()
        nbv0 = cntv[pl.ds(bucket, 16)]
        nb0 = nbv0[0]
        nbv1 = cntv[pl.ds(128 + bucket, 16)]
        nb1 = nbv1[0]
        nbt = nb0 + nb1
        nh = 2 * nbt

        def gather_fire(h, j):
            t = lax.shift_right_logical(h, 1)
            half = jnp.bitwise_and(h, 1)
            in0 = t < nb0
            bb = jnp.where(in0, t, t - nb0)
            pre = bb < _PRE
            po = bb * _BT + half * _HB

            @pl.when(in0 & pre)
            def _():
                pltpu.async_copy(hn_hbm.at[sp0.at[pl.ds(po, _HB)]],
                                 rbufs[j], gsem.at[j])

            @pl.when(jnp.logical_not(in0) & pre)
            def _():
                pltpu.async_copy(hn_hbm.at[sp1.at[pl.ds(po, _HB)]],
                                 rbufs[j], gsem.at[j])

            @pl.when(jnp.logical_not(pre))
            def _():
                ro = pl.multiple_of(jnp.where(in0, b0, b1) + bb * _BT, _BT)
                pltpu.sync_copy(brs_hbm.at[pl.ds(ro, _BT)], sfull)
                pltpu.sync_copy(brk_hbm.at[pl.ds(ro, _BT)], kfull)
                vcopy64(sfull, half * _HB, sbufs[j])
                vcopy64(kfull, half * _HB, kbufs[j])
                pltpu.async_copy(hn_hbm.at[sbufs[j]], rbufs[j], gsem.at[j])

        def scatter_fire(h, j):
            t = lax.shift_right_logical(h, 1)
            half = jnp.bitwise_and(h, 1)
            in0 = t < nb0
            bb = jnp.where(in0, t, t - nb0)
            pre = bb < _PRE
            po = bb * _BT + half * _HB

            @pl.when(in0 & pre)
            def _():
                vcopy64(kp0, po, kbufs[j])

            @pl.when(jnp.logical_not(in0) & pre)
            def _():
                vcopy64(kp1, po, kbufs[j])

            pltpu.async_copy(rbufs[j], acc.at[kbufs[j]], ssem.at[j], add=True)

        def it_body(it, _):
            for j in range(6):
                h = it * 6 + j
                jq = (j - 4) % 6

                @pl.when(h < nh)
                def _():
                    @pl.when(h >= 6)
                    def _():
                        wait_sca(j)
                    gather_fire(h, j)

                g4 = h - 4

                @pl.when((g4 >= 0) & (g4 < nh))
                def _():
                    wait_gat(jq)
                    scatter_fire(g4, jq)

            return 0

        nit = (nh + 9) // 6
        lax.fori_loop(0, nit, it_body, 0)
        for j in range(6):
            @pl.when(j < nh)
            def _():
                wait_sca(j)
        plsc.subcore_barrier()
        for rr in range(_R):
            wo = pl.multiple_of(rr * _N + bucket * _BKT + s * 128, 128)
            ao = pl.multiple_of(rr * _BKT + s * 128, 128)
            pltpu.sync_copy(acc.at[pl.ds(ao, 128)],
                            sout_hbm.at[pl.ds(wo, 128)])
        plsc.subcore_barrier()
        return 0

    lax.fori_loop(0, 8, bucket_body, 0)


def _segsum_call(hn128, brs, brk, cntf):
    mesh = plsc.VectorSubcoreMesh(core_axis_name="c", subcore_axis_name="s")
    f = pl.kernel(
        _seg_body,
        out_type=[
            jax.ShapeDtypeStruct((_R * _N, _W), jnp.float32),
        ],
        mesh=mesh,
        compiler_params=pltpu.CompilerParams(needs_layout_passes=False),
        scratch_types=[
            pltpu.VMEM_SHARED((_ACC_ROWS, _W), jnp.float32),
        ] + [pltpu.VMEM((_HB,), jnp.int32) for _ in range(12)]
          + [pltpu.VMEM((_HB, _W), jnp.float32) for _ in range(6)]
          + [pltpu.VMEM((_BT,), jnp.int32) for _ in range(2)]
          + [pltpu.VMEM((_PRE * _BT,), jnp.int32) for _ in range(4)]
          + [
            pltpu.VMEM((272,), jnp.int32),
            pltpu.SemaphoreType.DMA,
            pltpu.SemaphoreType.DMA((6,)),
            pltpu.SemaphoreType.DMA((6,)),
        ],
    )
    return f(hn128, brs, brk, cntf)


def _ln(h, s, b):
    m = jnp.mean(h, axis=-1, keepdims=True)
    v = jnp.mean((h - m) ** 2, axis=-1, keepdims=True)
    return (h - m) / jnp.sqrt(v + 1e-5) * s + b


def _prep_body(h_ref, s_ref, b_ref, out_ref):
    h = h_ref[...]
    t, c = h.shape
    hn = _ln(h, s_ref[...], b_ref[...])
    ones = jnp.ones((t, 1), jnp.float32)
    zeros = jnp.zeros((t, _W - c - 1), jnp.float32)
    out_ref[...] = jnp.concatenate([hn, ones, zeros], axis=1)


def _prep_call(h, ln_s, ln_b, tile):
    n, c = h.shape
    grid = (n // tile,)
    return pl.pallas_call(
        _prep_body,
        grid=grid,
        in_specs=[
            pl.BlockSpec((tile, c), lambda i: (i, 0)),
            pl.BlockSpec((1, c), lambda i: (0, 0)),
            pl.BlockSpec((1, c), lambda i: (0, 0)),
        ],
        out_specs=pl.BlockSpec((tile, _W), lambda i: (i, 0)),
        out_shape=jax.ShapeDtypeStruct((n, _W), jnp.float32),
    )(h, ln_s.reshape(1, c), ln_b.reshape(1, c))


def _block_body(r, c, hid, emit_next,
                h_ref, s_acc_ref,
                ln1s_ref, ln1b_ref, wg_ref, bg_ref, wrel_ref, brel_ref,
                wself_ref, bself_ref, wp_ref, bp_ref,
                ln2s_ref, ln2b_ref, w1_ref, b1_ref, w2_ref, b2_ref,
                nls_ref, nlb_ref,
                hout_ref, hn128_ref):
    h = h_ref[...]
    hn = _ln(h, ln1s_ref[...], ln1b_ref[...])
    logits = jnp.dot(hn, wg_ref[...], preferred_element_type=jnp.float32) + bg_ref[...]
    gmax = jnp.max(logits, axis=-1, keepdims=True)
    ge = jnp.exp(logits - gmax)
    gates = ge / jnp.sum(ge, axis=-1, keepdims=True)
    comb = jnp.dot(hn, wself_ref[...], preferred_element_type=jnp.float32)
    comb = comb + bself_ref[...]
    for rr in range(r):
        s_r = s_acc_ref[rr]
        cnt = s_r[:, c:c + 1]
        denom = jnp.maximum(cnt, 1.0)
        g_r = gates[:, rr:rr + 1]
        scaled = (g_r / denom) * s_r[:, :c]
        comb = comb + jnp.dot(scaled, wrel_ref[rr], preferred_element_type=jnp.float32)
        has = jnp.where(cnt > 0.0, 1.0, 0.0)
        comb = comb + (g_r * has) * brel_ref[rr:rr + 1, :]
    act = jax.nn.gelu(comb)
    out = jnp.dot(act, wp_ref[...], preferred_element_type=jnp.float32) + bp_ref[...]
    h1 = h + out
    h2 = _ln(h1, ln2s_ref[...], ln2b_ref[...])
    ffn = jax.nn.gelu(jnp.dot(h2, w1_ref[...], preferred_element_type=jnp.float32) + b1_ref[...])
    ffn = jnp.dot(ffn, w2_ref[...], preferred_element_type=jnp.float32) + b2_ref[...]
    hf = h1 + ffn
    hout_ref[...] = hf
    if emit_next:
        t = hf.shape[0]
        hn_next = _ln(hf, nls_ref[...], nlb_ref[...])
        ones = jnp.ones((t, 1), jnp.float32)
        zeros = jnp.zeros((t, _W - c - 1), jnp.float32)
        hn128_ref[...] = jnp.concatenate([hn_next, ones, zeros], axis=1)
    else:
        hn128_ref[...] = jnp.zeros_like(hn128_ref)


def _block_call(h, s_acc, blk, next_ln, tile):
    n, c = h.shape
    r = blk['b_rel'].shape[0]
    hid = blk['W1'].shape[1]
    emit_next = next_ln is not None
    nls = (next_ln[0] if emit_next else blk['ln1_s']).reshape(1, c)
    nlb = (next_ln[1] if emit_next else blk['ln1_b']).reshape(1, c)
    grid = (n // tile,)
    full = lambda a: pl.BlockSpec(a.shape, lambda i: (0,) * a.ndim)
    args = [
        blk['ln1_s'].reshape(1, c), blk['ln1_b'].reshape(1, c),
        blk['Wg'], blk['bg'].reshape(1, r),
        blk['W_rel'], blk['b_rel'],
        blk['W_self'], blk['b_self'].reshape(1, c),
        blk['Wp'], blk['bp'].reshape(1, c),
        blk['ln2_s'].reshape(1, c), blk['ln2_b'].reshape(1, c),
        blk['W1'], blk['b1'].reshape(1, hid),
        blk['W2'], blk['b2'].reshape(1, c),
        nls, nlb,
    ]
    in_specs = [
        pl.BlockSpec((tile, c), lambda i: (i, 0)),
        pl.BlockSpec((r, tile, _W), lambda i: (0, i, 0)),
    ] + [full(a) for a in args]
    body = functools.partial(_block_body, r, c, hid, emit_next)
    return pl.pallas_call(
        body,
        grid=grid,
        in_specs=in_specs,
        out_specs=[
            pl.BlockSpec((tile, c), lambda i: (i, 0)),
            pl.BlockSpec((tile, _W), lambda i: (i, 0)),
        ],
        out_shape=[
            jax.ShapeDtypeStruct((n, c), jnp.float32),
            jax.ShapeDtypeStruct((n, _W), jnp.float32),
        ],
    )(h, s_acc, *args)


def kernel(x, params, edge_index, edge_relation):
    b, l, c = x.shape
    n = b * l
    blocks = params['blocks']
    r = blocks[0]['b_rel'].shape[0]
    assert n == _N and r == _R and edge_index.shape[1] == _E
    src = edge_index[0]
    dst = edge_index[1]
    tile = 1024

    brs, brk, cntf = _bucket_call(src, dst, edge_relation)
    h = x.reshape(n, c)
    hn128 = _prep_call(h, blocks[0]['ln1_s'], blocks[0]['ln1_b'], tile)
    for bi, blk in enumerate(blocks):
        (s_flat,) = _segsum_call(hn128, brs, brk, cntf)
        s_acc = s_flat.reshape(r, n, _W)
        nxt = None
        if bi + 1 < len(blocks):
            nxt = (blocks[bi + 1]['ln1_s'], blocks[bi + 1]['ln1_b'])
        h, hn128 = _block_call(h, s_acc, blk, nxt, tile)
    return h.reshape(b, l, c)


# speedup vs baseline: 1.0023x; 1.0023x over previous
import functools

import jax
import jax.numpy as jnp
from jax import lax
from jax.experimental import pallas as pl
from jax.experimental.pallas import tpu as pltpu
from jax.experimental.pallas import tpu_sc as plsc

_E = 1048576
_N = 32768
_R = 4
_NW = 32
_EPW = _E // _NW
_CH = 2048
_NCH = _EPW // _CH
_NBUK = 16
_BKT = _N // _NBUK
_BT = 128
_STG = _BT + 16
_CAPB = _EPW + _BT
_DUMP = _R * _BKT
_ACC_ROWS = _DUMP + 256
_W = 128


def _bucket_body(src_hbm, dst_hbm, rel_hbm, brs_hbm, brk_hbm, cnt_hbm,
                 sbuf, dbuf, rbuf, stgs, stgk, cntv):
    c = lax.axis_index("c")
    s = lax.axis_index("s")
    wid = 2 * s + c
    ebase = wid * _EPW

    def chunk_body(ch, carry):
        base = pl.multiple_of(ebase + ch * _CH, _CH)
        pltpu.sync_copy(src_hbm.at[pl.ds(base, _CH)], sbuf)
        pltpu.sync_copy(dst_hbm.at[pl.ds(base, _CH)], dbuf)
        pltpu.sync_copy(rel_hbm.at[pl.ds(base, _CH)], rbuf)

        def vec_body(i, cr):
            fills, offs = cr[:_NBUK], cr[_NBUK:]
            sv = sbuf[pl.ds(i * 16, 16)]
            dv = dbuf[pl.ds(i * 16, 16)]
            rv = rbuf[pl.ds(i * 16, 16)]
            bkt = lax.shift_right_logical(dv, 11)
            key = jnp.bitwise_or(lax.shift_left(rv, 11),
                                 jnp.bitwise_and(dv, _BKT - 1))
            nf, no = [], []
            for b in range(_NBUK):
                fill, off = fills[b], offs[b]
                m = bkt == b
                cb = jnp.sum(m.astype(jnp.int32))
                plsc.store_compressed(stgs.at[pl.ds(b * _STG + fill, 16)], sv, mask=m)
                plsc.store_compressed(stgk.at[pl.ds(b * _STG + fill, 16)], key, mask=m)
                fill = fill + cb

                def flush(fl, of, b=b):
                    ro = pl.multiple_of((wid * _NBUK + b) * _CAPB + of * _BT, _BT)
                    pltpu.sync_copy(stgs.at[pl.ds(b * _STG, _BT)],
                                    brs_hbm.at[pl.ds(ro, _BT)])
                    pltpu.sync_copy(stgk.at[pl.ds(b * _STG, _BT)],
                                    brk_hbm.at[pl.ds(ro, _BT)])
                    los = stgs[pl.ds(b * _STG + _BT, 16)]
                    lok = stgk[pl.ds(b * _STG + _BT, 16)]
                    stgs[pl.ds(b * _STG, 16)] = los
                    stgk[pl.ds(b * _STG, 16)] = lok
                    return fl - _BT, of + 1

                fill, off = lax.cond(fill >= _BT, flush,
                                     lambda fl, of: (fl, of), fill, off)
                nf.append(fill)
                no.append(off)
            return tuple(nf) + tuple(no)

        return lax.fori_loop(0, _CH // 16, vec_body, carry)

    carry = (jnp.int32(0),) * (2 * _NBUK)
    carry = lax.fori_loop(0, _NCH, chunk_body, carry)
    fills, offs = carry[:_NBUK], carry[_NBUK:]
    dsrc = jnp.zeros((16,), jnp.int32)
    dkey = jnp.full((16,), _DUMP, jnp.int32)
    for b in range(_NBUK):
        fill, off = fills[b], offs[b]

        def final_flush(fl, of, b=b):
            npad = (_BT - fl + 15) // 16

            def pad_body(j, _):
                stgs[pl.ds(b * _STG + fl + j * 16, 16)] = dsrc
                stgk[pl.ds(b * _STG + fl + j * 16, 16)] = dkey
                return 0

            lax.fori_loop(0, npad, pad_body, 0)
            ro = pl.multiple_of((wid * _NBUK + b) * _CAPB + of * _BT, _BT)
            pltpu.sync_copy(stgs.at[pl.ds(b * _STG, _BT)],
                            brs_hbm.at[pl.ds(ro, _BT)])
            pltpu.sync_copy(stgk.at[pl.ds(b * _STG, _BT)],
                            brk_hbm.at[pl.ds(ro, _BT)])
            return of + 1

        off = lax.cond(fill > 0, final_flush, lambda fl, of: of, fill, off)
        offs = offs[:b] + (off,) + offs[b + 1:]
    lanes = lax.iota(jnp.int32, 16)
    cvec = jnp.zeros((16,), jnp.int32)
    for b in range(_NBUK):
        cvec = jnp.where(lanes == b, offs[b], cvec)
    cntv[...] = cvec
    co = pl.multiple_of(wid * 128, 128)
    pltpu.sync_copy(cntv, cnt_hbm.at[pl.ds(co, 16)])


def _bucket_call(src, dst, rel):
    mesh = plsc.VectorSubcoreMesh(core_axis_name="c", subcore_axis_name="s")
    f = pl.kernel(
        _bucket_body,
        out_type=[
            jax.ShapeDtypeStruct((_NW * _NBUK * _CAPB,), jnp.int32),
            jax.ShapeDtypeStruct((_NW * _NBUK * _CAPB,), jnp.int32),
            jax.ShapeDtypeStruct((_NW * 128,), jnp.int32),
        ],
        mesh=mesh,
        compiler_params=pltpu.CompilerParams(needs_layout_passes=False),
        scratch_types=[
            pltpu.VMEM((_CH,), jnp.int32),
            pltpu.VMEM((_CH,), jnp.int32),
            pltpu.VMEM((_CH,), jnp.int32),
            pltpu.VMEM((_NBUK * _STG,), jnp.int32),
            pltpu.VMEM((_NBUK * _STG,), jnp.int32),
            pltpu.VMEM((_NBUK,), jnp.int32),
        ],
    )
    return f(src, dst, rel)


_PRE = 20
_HB = 64


def _seg_body(hn_hbm, brs_hbm, brk_hbm, cntf_hbm, sout_hbm,
              acc, s0, s1, s2, s3, s4, s5, k0, k1, k2, k3, k4, k5,
              r0, r1, r2, r3, r4, r5, sfull, kfull,
              sp0, sp1, kp0, kp1, cntv, psem, gsem, ssem):
    c = lax.axis_index("c")
    s = lax.axis_index("s")
    sbufs = (s0, s1, s2, s3, s4, s5)
    kbufs = (k0, k1, k2, k3, k4, k5)
    rbufs = (r0, r1, r2, r3, r4, r5)
    st0 = 2 * s
    co = pl.multiple_of(st0 * 128, 128)
    pltpu.sync_copy(cntf_hbm.at[pl.ds(co, 256)], cntv.at[pl.ds(0, 256)])
    zv = jnp.zeros((16,), jnp.float32)

    def zrow(i, _):
        for j in range(_W // 16):
            r0[i, pl.ds(j * 16, 16)] = zv
        return 0

    def wait_gat(j):
        pltpu.make_async_copy(hn_hbm.at[sbufs[j]], rbufs[j], gsem.at[j]).wait()

    def wait_sca(j):
        pltpu.make_async_copy(rbufs[j], acc.at[kbufs[j]], ssem.at[j]).wait()

    def vcopy64(srcref, soff, dstref):
        for q in range(4):
            dstref[pl.ds(q * 16, 16)] = srcref[pl.ds(soff + q * 16, 16)]

    def bucket_body(bl, _):
        bucket = c * 8 + bl
        b0 = pl.multiple_of((st0 * _NBUK + bucket) * _CAPB, 128)
        b1 = pl.multiple_of(((st0 + 1) * _NBUK + bucket) * _CAPB, 128)
        pltpu.async_copy(brs_hbm.at[pl.ds(b0, _PRE * _BT)], sp0, psem)
        pltpu.async_copy(brk_hbm.at[pl.ds(b0, _PRE * _BT)], kp0, psem)
        pltpu.async_copy(brs_hbm.at[pl.ds(b1, _PRE * _BT)], sp1, psem)
        pltpu.async_copy(brk_hbm.at[pl.ds(b1, _PRE * _BT)], kp1, psem)
        lax.fori_loop(0, _HB, zrow, 0)
        row0 = s * (_ACC_ROWS // 16)
        for q in range(8):
            pltpu.sync_copy(r0, acc.at[pl.ds(row0 + q * _HB, _HB)])
        pltpu.sync_copy(r0.at[pl.ds(0, 16)], acc.at[pl.ds(row0 + 8 * _HB, 16)])
        for pb in (sp0, kp0, sp1, kp1):
            pltpu.make_async_copy(brs_hbm.at[pl.ds(0, _PRE * _BT)], pb, psem).wait()
        plsc.subcore_barrier()
        nbv0 = cntv[pl.ds(bucket, 16)]
        nb0 = nbv0[0]
        nbv1 = cntv[pl.ds(128 + bucket, 16)]
        nb1 = nbv1[0]
        nbt = nb0 + nb1
        nh = 2 * nbt

        def gather_fire(h, j):
            t = lax.shift_right_logical(h, 1)
            half = jnp.bitwise_and(h, 1)
            in0 = t < nb0
            bb = jnp.where(in0, t, t - nb0)
            pre = bb < _PRE
            po = bb * _BT + half * _HB

            @pl.when(in0 & pre)
            def _():
                pltpu.async_copy(hn_hbm.at[sp0.at[pl.ds(po, _HB)]],
                                 rbufs[j], gsem.at[j])

            @pl.when(jnp.logical_not(in0) & pre)
            def _():
                pltpu.async_copy(hn_hbm.at[sp1.at[pl.ds(po, _HB)]],
                                 rbufs[j], gsem.at[j])

            @pl.when(jnp.logical_not(pre))
            def _():
                ro = pl.multiple_of(jnp.where(in0, b0, b1) + bb * _BT, _BT)
                pltpu.sync_copy(brs_hbm.at[pl.ds(ro, _BT)], sfull)
                pltpu.sync_copy(brk_hbm.at[pl.ds(ro, _BT)], kfull)
                vcopy64(sfull, half * _HB, sbufs[j])
                vcopy64(kfull, half * _HB, kbufs[j])
                pltpu.async_copy(hn_hbm.at[sbufs[j]], rbufs[j], gsem.at[j])

        def scatter_fire(h, j):
            t = lax.shift_right_logical(h, 1)
            half = jnp.bitwise_and(h, 1)
            in0 = t < nb0
            bb = jnp.where(in0, t, t - nb0)
            pre = bb < _PRE
            po = bb * _BT + half * _HB

            @pl.when(in0 & pre)
            def _():
                vcopy64(kp0, po, kbufs[j])

            @pl.when(jnp.logical_not(in0) & pre)
            def _():
                vcopy64(kp1, po, kbufs[j])

            pltpu.async_copy(rbufs[j], acc.at[kbufs[j]], ssem.at[j], add=True)

        def it_body(it, _):
            for j in range(6):
                h = it * 6 + j
                jq = (j - 4) % 6

                @pl.when(h < nh)
                def _():
                    @pl.when(h >= 6)
                    def _():
                        wait_sca(j)
                    gather_fire(h, j)

                g4 = h - 4

                @pl.when((g4 >= 0) & (g4 < nh))
                def _():
                    wait_gat(jq)
                    scatter_fire(g4, jq)

            return 0

        nit = (nh + 9) // 6
        lax.fori_loop(0, nit, it_body, 0)
        for j in range(6):
            @pl.when(j < nh)
            def _():
                wait_sca(j)
        plsc.subcore_barrier()
        for rr in range(_R):
            wo = pl.multiple_of(rr * _N + bucket * _BKT + s * 128, 128)
            ao = pl.multiple_of(rr * _BKT + s * 128, 128)
            pltpu.sync_copy(acc.at[pl.ds(ao, 128)],
                            sout_hbm.at[pl.ds(wo, 128)])
        plsc.subcore_barrier()
        return 0

    lax.fori_loop(0, 8, bucket_body, 0)


def _segsum_call(hn128, brs, brk, cntf):
    mesh = plsc.VectorSubcoreMesh(core_axis_name="c", subcore_axis_name="s")
    f = pl.kernel(
        _seg_body,
        out_type=[
            jax.ShapeDtypeStruct((_R * _N, _W), jnp.float32),
        ],
        mesh=mesh,
        compiler_params=pltpu.CompilerParams(needs_layout_passes=False),
        scratch_types=[
            pltpu.VMEM_SHARED((_ACC_ROWS, _W), jnp.float32),
        ] + [pltpu.VMEM((_HB,), jnp.int32) for _ in range(12)]
          + [pltpu.VMEM((_HB, _W), jnp.float32) for _ in range(6)]
          + [pltpu.VMEM((_BT,), jnp.int32) for _ in range(2)]
          + [pltpu.VMEM((_PRE * _BT,), jnp.int32) for _ in range(4)]
          + [
            pltpu.VMEM((272,), jnp.int32),
            pltpu.SemaphoreType.DMA,
            pltpu.SemaphoreType.DMA((6,)),
            pltpu.SemaphoreType.DMA((6,)),
        ],
    )
    return f(hn128, brs, brk, cntf)


def _bfdot(a, w):
    return jnp.dot(a.astype(jnp.bfloat16), w.astype(jnp.bfloat16),
                   preferred_element_type=jnp.float32)


def _ln(h, s, b):
    m = jnp.mean(h, axis=-1, keepdims=True)
    v = jnp.mean((h - m) ** 2, axis=-1, keepdims=True)
    return (h - m) / jnp.sqrt(v + 1e-5) * s + b


def _prep_body(h_ref, s_ref, b_ref, out_ref):
    h = h_ref[...]
    t, c = h.shape
    hn = _ln(h, s_ref[...], b_ref[...])
    ones = jnp.ones((t, 1), jnp.float32)
    zeros = jnp.zeros((t, _W - c - 1), jnp.float32)
    out_ref[...] = jnp.concatenate([hn, ones, zeros], axis=1)


def _prep_call(h, ln_s, ln_b, tile):
    n, c = h.shape
    grid = (n // tile,)
    return pl.pallas_call(
        _prep_body,
        grid=grid,
        in_specs=[
            pl.BlockSpec((tile, c), lambda i: (i, 0)),
            pl.BlockSpec((1, c), lambda i: (0, 0)),
            pl.BlockSpec((1, c), lambda i: (0, 0)),
        ],
        out_specs=pl.BlockSpec((tile, _W), lambda i: (i, 0)),
        out_shape=jax.ShapeDtypeStruct((n, _W), jnp.float32),
    )(h, ln_s.reshape(1, c), ln_b.reshape(1, c))


def _block_body(r, c, hid, emit_next,
                h_ref, s_acc_ref,
                ln1s_ref, ln1b_ref, wg_ref, bg_ref, wrel_ref, brel_ref,
                wself_ref, bself_ref, wp_ref, bp_ref,
                ln2s_ref, ln2b_ref, w1_ref, b1_ref, w2_ref, b2_ref,
                nls_ref, nlb_ref,
                hout_ref, hn128_ref):
    h = h_ref[...]
    hn = _ln(h, ln1s_ref[...], ln1b_ref[...])
    logits = jnp.dot(hn, wg_ref[...], preferred_element_type=jnp.float32) + bg_ref[...]
    gmax = jnp.max(logits, axis=-1, keepdims=True)
    ge = jnp.exp(logits - gmax)
    gates = ge / jnp.sum(ge, axis=-1, keepdims=True)
    comb = _bfdot(hn, wself_ref[...])
    comb = comb + bself_ref[...]
    for rr in range(r):
        s_r = s_acc_ref[rr]
        cnt = s_r[:, c:c + 1]
        denom = jnp.maximum(cnt, 1.0)
        g_r = gates[:, rr:rr + 1]
        scaled = (g_r / denom) * s_r[:, :c]
        comb = comb + _bfdot(scaled, wrel_ref[rr])
        has = jnp.where(cnt > 0.0, 1.0, 0.0)
        comb = comb + (g_r * has) * brel_ref[rr:rr + 1, :]
    act = jax.nn.gelu(comb)
    out = _bfdot(act, wp_ref[...]) + bp_ref[...]
    h1 = h + out
    h2 = _ln(h1, ln2s_ref[...], ln2b_ref[...])
    ffn = jax.nn.gelu(_bfdot(h2, w1_ref[...]) + b1_ref[...])
    ffn = _bfdot(ffn, w2_ref[...]) + b2_ref[...]
    hf = h1 + ffn
    hout_ref[...] = hf
    if emit_next:
        t = hf.shape[0]
        hn_next = _ln(hf, nls_ref[...], nlb_ref[...])
        ones = jnp.ones((t, 1), jnp.float32)
        zeros = jnp.zeros((t, _W - c - 1), jnp.float32)
        hn128_ref[...] = jnp.concatenate([hn_next, ones, zeros], axis=1)
    else:
        hn128_ref[...] = jnp.zeros_like(hn128_ref)


def _block_call(h, s_acc, blk, next_ln, tile):
    n, c = h.shape
    r = blk['b_rel'].shape[0]
    hid = blk['W1'].shape[1]
    emit_next = next_ln is not None
    nls = (next_ln[0] if emit_next else blk['ln1_s']).reshape(1, c)
    nlb = (next_ln[1] if emit_next else blk['ln1_b']).reshape(1, c)
    grid = (n // tile,)
    full = lambda a: pl.BlockSpec(a.shape, lambda i: (0,) * a.ndim)
    args = [
        blk['ln1_s'].reshape(1, c), blk['ln1_b'].reshape(1, c),
        blk['Wg'], blk['bg'].reshape(1, r),
        blk['W_rel'], blk['b_rel'],
        blk['W_self'], blk['b_self'].reshape(1, c),
        blk['Wp'], blk['bp'].reshape(1, c),
        blk['ln2_s'].reshape(1, c), blk['ln2_b'].reshape(1, c),
        blk['W1'], blk['b1'].reshape(1, hid),
        blk['W2'], blk['b2'].reshape(1, c),
        nls, nlb,
    ]
    in_specs = [
        pl.BlockSpec((tile, c), lambda i: (i, 0)),
        pl.BlockSpec((r, tile, _W), lambda i: (0, i, 0)),
    ] + [full(a) for a in args]
    body = functools.partial(_block_body, r, c, hid, emit_next)
    return pl.pallas_call(
        body,
        grid=grid,
        in_specs=in_specs,
        out_specs=[
            pl.BlockSpec((tile, c), lambda i: (i, 0)),
            pl.BlockSpec((tile, _W), lambda i: (i, 0)),
        ],
        out_shape=[
            jax.ShapeDtypeStruct((n, c), jnp.float32),
            jax.ShapeDtypeStruct((n, _W), jnp.float32),
        ],
    )(h, s_acc, *args)


def kernel(x, params, edge_index, edge_relation):
    b, l, c = x.shape
    n = b * l
    blocks = params['blocks']
    r = blocks[0]['b_rel'].shape[0]
    assert n == _N and r == _R and edge_index.shape[1] == _E
    src = edge_index[0]
    dst = edge_index[1]
    tile = 1024

    brs, brk, cntf = _bucket_call(src, dst, edge_relation)
    h = x.reshape(n, c)
    hn128 = _prep_call(h, blocks[0]['ln1_s'], blocks[0]['ln1_b'], tile)
    for bi, blk in enumerate(blocks):
        (s_flat,) = _segsum_call(hn128, brs, brk, cntf)
        s_acc = s_flat.reshape(r, n, _W)
        nxt = None
        if bi + 1 < len(blocks):
            nxt = (blocks[bi + 1]['ln1_s'], blocks[bi + 1]['ln1_b'])
        h, hn128 = _block_call(h, s_acc, blk, nxt, tile)
    return h.reshape(b, l, c)


# speedup vs baseline: 1.1488x; 1.1461x over previous
import functools

import jax
import jax.numpy as jnp
from jax import lax
from jax.experimental import pallas as pl
from jax.experimental.pallas import tpu as pltpu
from jax.experimental.pallas import tpu_sc as plsc

_E = 1048576
_N = 32768
_R = 4
_NW = 32
_EPW = _E // _NW
_CH = 2048
_NCH = _EPW // _CH
_NBUK = 16
_BKT = _N // _NBUK
_BT = 128
_STG = _BT + 16
_CAPB = _EPW + _BT
_DUMP = _R * _BKT
_ACC_ROWS = _DUMP + 256
_W = 128


def _bucket_body(src_hbm, dst_hbm, rel_hbm, brs_hbm, brk_hbm, cnt_hbm,
                 sbuf, dbuf, rbuf, stgs, stgk, cntv):
    c = lax.axis_index("c")
    s = lax.axis_index("s")
    wid = 2 * s + c
    ebase = wid * _EPW

    def chunk_body(ch, carry):
        base = pl.multiple_of(ebase + ch * _CH, _CH)
        pltpu.sync_copy(src_hbm.at[pl.ds(base, _CH)], sbuf)
        pltpu.sync_copy(dst_hbm.at[pl.ds(base, _CH)], dbuf)
        pltpu.sync_copy(rel_hbm.at[pl.ds(base, _CH)], rbuf)

        def vec_body(i, cr):
            fills, offs = cr[:_NBUK], cr[_NBUK:]
            sv = sbuf[pl.ds(i * 16, 16)]
            dv = dbuf[pl.ds(i * 16, 16)]
            rv = rbuf[pl.ds(i * 16, 16)]
            bkt = lax.shift_right_logical(dv, 11)
            key = jnp.bitwise_or(lax.shift_left(rv, 11),
                                 jnp.bitwise_and(dv, _BKT - 1))
            nf, no = [], []
            for b in range(_NBUK):
                fill, off = fills[b], offs[b]
                m = bkt == b
                cb = jnp.sum(m.astype(jnp.int32))
                plsc.store_compressed(stgs.at[pl.ds(b * _STG + fill, 16)], sv, mask=m)
                plsc.store_compressed(stgk.at[pl.ds(b * _STG + fill, 16)], key, mask=m)
                fill = fill + cb

                def flush(fl, of, b=b):
                    ro = pl.multiple_of((wid * _NBUK + b) * _CAPB + of * _BT, _BT)
                    pltpu.sync_copy(stgs.at[pl.ds(b * _STG, _BT)],
                                    brs_hbm.at[pl.ds(ro, _BT)])
                    pltpu.sync_copy(stgk.at[pl.ds(b * _STG, _BT)],
                                    brk_hbm.at[pl.ds(ro, _BT)])
                    los = stgs[pl.ds(b * _STG + _BT, 16)]
                    lok = stgk[pl.ds(b * _STG + _BT, 16)]
                    stgs[pl.ds(b * _STG, 16)] = los
                    stgk[pl.ds(b * _STG, 16)] = lok
                    return fl - _BT, of + 1

                fill, off = lax.cond(fill >= _BT, flush,
                                     lambda fl, of: (fl, of), fill, off)
                nf.append(fill)
                no.append(off)
            return tuple(nf) + tuple(no)

        return lax.fori_loop(0, _CH // 16, vec_body, carry)

    carry = (jnp.int32(0),) * (2 * _NBUK)
    carry = lax.fori_loop(0, _NCH, chunk_body, carry)
    fills, offs = carry[:_NBUK], carry[_NBUK:]
    dsrc = jnp.zeros((16,), jnp.int32)
    dkey = jnp.full((16,), _DUMP, jnp.int32)
    for b in range(_NBUK):
        fill, off = fills[b], offs[b]

        def final_flush(fl, of, b=b):
            npad = (_BT - fl + 15) // 16

            def pad_body(j, _):
                stgs[pl.ds(b * _STG + fl + j * 16, 16)] = dsrc
                stgk[pl.ds(b * _STG + fl + j * 16, 16)] = dkey
                return 0

            lax.fori_loop(0, npad, pad_body, 0)
            ro = pl.multiple_of((wid * _NBUK + b) * _CAPB + of * _BT, _BT)
            pltpu.sync_copy(stgs.at[pl.ds(b * _STG, _BT)],
                            brs_hbm.at[pl.ds(ro, _BT)])
            pltpu.sync_copy(stgk.at[pl.ds(b * _STG, _BT)],
                            brk_hbm.at[pl.ds(ro, _BT)])
            return of + 1

        off = lax.cond(fill > 0, final_flush, lambda fl, of: of, fill, off)
        offs = offs[:b] + (off,) + offs[b + 1:]
    lanes = lax.iota(jnp.int32, 16)
    cvec = jnp.zeros((16,), jnp.int32)
    for b in range(_NBUK):
        cvec = jnp.where(lanes == b, offs[b], cvec)
    cntv[...] = cvec
    co = pl.multiple_of(wid * 128, 128)
    pltpu.sync_copy(cntv, cnt_hbm.at[pl.ds(co, 16)])


def _bucket_call(src, dst, rel):
    mesh = plsc.VectorSubcoreMesh(core_axis_name="c", subcore_axis_name="s")
    f = pl.kernel(
        _bucket_body,
        out_type=[
            jax.ShapeDtypeStruct((_NW * _NBUK * _CAPB,), jnp.int32),
            jax.ShapeDtypeStruct((_NW * _NBUK * _CAPB,), jnp.int32),
            jax.ShapeDtypeStruct((_NW * 128,), jnp.int32),
        ],
        mesh=mesh,
        compiler_params=pltpu.CompilerParams(needs_layout_passes=False),
        scratch_types=[
            pltpu.VMEM((_CH,), jnp.int32),
            pltpu.VMEM((_CH,), jnp.int32),
            pltpu.VMEM((_CH,), jnp.int32),
            pltpu.VMEM((_NBUK * _STG,), jnp.int32),
            pltpu.VMEM((_NBUK * _STG,), jnp.int32),
            pltpu.VMEM((_NBUK,), jnp.int32),
        ],
    )
    return f(src, dst, rel)


_PRE = 20
_HB = 64


def _seg_body(hn_hbm, brs_hbm, brk_hbm, cntf_hbm, sout_hbm,
              acc, s0, s1, s2, s3, k0, k1, k2, k3,
              g0, g1, g2, g3, r0, r1, r2, r3, sfull, kfull,
              sp0, sp1, kp0, kp1, cntv, psem, gsem, ssem):
    c = lax.axis_index("c")
    s = lax.axis_index("s")
    sbufs = (s0, s1, s2, s3)
    kbufs = (k0, k1, k2, k3)
    gbufs = (g0, g1, g2, g3)
    rbufs = (r0, r1, r2, r3)
    st0 = 2 * s
    co = pl.multiple_of(st0 * 128, 128)
    pltpu.sync_copy(cntf_hbm.at[pl.ds(co, 256)], cntv.at[pl.ds(0, 256)])
    zv = jnp.zeros((16,), jnp.float32)

    def zrow(i, _):
        for j in range(_W // 16):
            r0[i, pl.ds(j * 16, 16)] = zv
        return 0

    def wait_gat(j):
        pltpu.make_async_copy(hn_hbm.at[sbufs[j]], gbufs[j], gsem.at[j]).wait()

    def wait_sca(j):
        pltpu.make_async_copy(rbufs[j], acc.at[kbufs[j]], ssem.at[j]).wait()

    def vcopy64(srcref, soff, dstref):
        for q in range(4):
            dstref[pl.ds(q * 16, 16)] = srcref[pl.ds(soff + q * 16, 16)]

    def convert(j):
        gb = gbufs[j]
        rb = rbufs[j]

        def crow(i, _):
            for g in range(4):
                v = jnp.reshape(gb[i, pl.ds(g * 16, 16)], (16,))
                vb = plsc.bitcast(v, jnp.bfloat16)
                a, bq = plsc.unpack(vb, format=plsc.PackFormat.INTERLEAVED)
                rb[i, pl.ds(g * 32, 16)] = a
                rb[i, pl.ds(g * 32 + 16, 16)] = bq
            return 0

        lax.fori_loop(0, _HB, crow, 0)

    def bucket_body(bl, _):
        bucket = c * 8 + bl
        b0 = pl.multiple_of((st0 * _NBUK + bucket) * _CAPB, 128)
        b1 = pl.multiple_of(((st0 + 1) * _NBUK + bucket) * _CAPB, 128)
        pltpu.async_copy(brs_hbm.at[pl.ds(b0, _PRE * _BT)], sp0, psem)
        pltpu.async_copy(brk_hbm.at[pl.ds(b0, _PRE * _BT)], kp0, psem)
        pltpu.async_copy(brs_hbm.at[pl.ds(b1, _PRE * _BT)], sp1, psem)
        pltpu.async_copy(brk_hbm.at[pl.ds(b1, _PRE * _BT)], kp1, psem)
        lax.fori_loop(0, _HB, zrow, 0)
        row0 = s * (_ACC_ROWS // 16)
        for q in range(8):
            pltpu.sync_copy(r0, acc.at[pl.ds(row0 + q * _HB, _HB)])
        pltpu.sync_copy(r0.at[pl.ds(0, 16)], acc.at[pl.ds(row0 + 8 * _HB, 16)])
        for pb in (sp0, kp0, sp1, kp1):
            pltpu.make_async_copy(brs_hbm.at[pl.ds(0, _PRE * _BT)], pb, psem).wait()
        plsc.subcore_barrier()
        nbv0 = cntv[pl.ds(bucket, 16)]
        nb0 = nbv0[0]
        nbv1 = cntv[pl.ds(128 + bucket, 16)]
        nb1 = nbv1[0]
        nbt = nb0 + nb1
        nh = 2 * nbt

        def gather_fire(h, j):
            t = lax.shift_right_logical(h, 1)
            half = jnp.bitwise_and(h, 1)
            in0 = t < nb0
            bb = jnp.where(in0, t, t - nb0)
            pre = bb < _PRE
            po = bb * _BT + half * _HB

            @pl.when(in0 & pre)
            def _():
                pltpu.async_copy(hn_hbm.at[sp0.at[pl.ds(po, _HB)]],
                                 gbufs[j], gsem.at[j])

            @pl.when(jnp.logical_not(in0) & pre)
            def _():
                pltpu.async_copy(hn_hbm.at[sp1.at[pl.ds(po, _HB)]],
                                 gbufs[j], gsem.at[j])

            @pl.when(jnp.logical_not(pre))
            def _():
                ro = pl.multiple_of(jnp.where(in0, b0, b1) + bb * _BT, _BT)
                pltpu.sync_copy(brs_hbm.at[pl.ds(ro, _BT)], sfull)
                pltpu.sync_copy(brk_hbm.at[pl.ds(ro, _BT)], kfull)
                vcopy64(sfull, half * _HB, sbufs[j])
                vcopy64(kfull, half * _HB, kbufs[j])
                pltpu.async_copy(hn_hbm.at[sbufs[j]], gbufs[j], gsem.at[j])

        def scatter_fire(h, j):
            t = lax.shift_right_logical(h, 1)
            half = jnp.bitwise_and(h, 1)
            in0 = t < nb0
            bb = jnp.where(in0, t, t - nb0)
            pre = bb < _PRE
            po = bb * _BT + half * _HB

            @pl.when(in0 & pre)
            def _():
                vcopy64(kp0, po, kbufs[j])

            @pl.when(jnp.logical_not(in0) & pre)
            def _():
                vcopy64(kp1, po, kbufs[j])

            pltpu.async_copy(rbufs[j], acc.at[kbufs[j]], ssem.at[j], add=True)

        def it_body(it, _):
            for j in range(4):
                h = it * 4 + j
                jq = (j - 2) % 4

                @pl.when(h < nh)
                def _():
                    @pl.when(h >= 4)
                    def _():
                        wait_sca(j)
                    gather_fire(h, j)

                g2 = h - 2

                @pl.when((g2 >= 0) & (g2 < nh))
                def _():
                    wait_gat(jq)
                    convert(jq)
                    scatter_fire(g2, jq)

            return 0

        nit = (nh + 5) // 4
        lax.fori_loop(0, nit, it_body, 0)
        for j in range(4):
            @pl.when(j < nh)
            def _():
                wait_sca(j)
        plsc.subcore_barrier()
        for rr in range(_R):
            wo = pl.multiple_of(rr * _N + bucket * _BKT + s * 128, 128)
            ao = pl.multiple_of(rr * _BKT + s * 128, 128)
            pltpu.sync_copy(acc.at[pl.ds(ao, 128)],
                            sout_hbm.at[pl.ds(wo, 128)])
        plsc.subcore_barrier()
        return 0

    lax.fori_loop(0, 8, bucket_body, 0)


def _segsum_call(hn128, brs, brk, cntf):
    mesh = plsc.VectorSubcoreMesh(core_axis_name="c", subcore_axis_name="s")
    f = pl.kernel(
        _seg_body,
        out_type=[
            jax.ShapeDtypeStruct((_R * _N, _W), jnp.float32),
        ],
        mesh=mesh,
        compiler_params=pltpu.CompilerParams(needs_layout_passes=False,
                                             use_tc_tiling_on_sc=False),
        scratch_types=[
            pltpu.VMEM_SHARED((_ACC_ROWS, _W), jnp.float32),
        ] + [pltpu.VMEM((_HB,), jnp.int32) for _ in range(8)]
          + [pltpu.VMEM((_HB, _W // 2), jnp.int32) for _ in range(4)]
          + [pltpu.VMEM((_HB, _W), jnp.float32) for _ in range(4)]
          + [pltpu.VMEM((_BT,), jnp.int32) for _ in range(2)]
          + [pltpu.VMEM((_PRE * _BT,), jnp.int32) for _ in range(4)]
          + [
            pltpu.VMEM((272,), jnp.int32),
            pltpu.SemaphoreType.DMA,
            pltpu.SemaphoreType.DMA((4,)),
            pltpu.SemaphoreType.DMA((4,)),
        ],
    )
    return f(hn128, brs, brk, cntf)


def _bfdot(a, w):
    return jnp.dot(a.astype(jnp.bfloat16), w.astype(jnp.bfloat16),
                   preferred_element_type=jnp.float32)


def _ln(h, s, b):
    m = jnp.mean(h, axis=-1, keepdims=True)
    v = jnp.mean((h - m) ** 2, axis=-1, keepdims=True)
    return (h - m) / jnp.sqrt(v + 1e-5) * s + b


def _prep_body(h_ref, s_ref, b_ref, out_ref):
    h = h_ref[...]
    t, c = h.shape
    hn = _ln(h, s_ref[...], b_ref[...])
    ones = jnp.ones((t, 1), jnp.float32)
    zeros = jnp.zeros((t, _W - c - 1), jnp.float32)
    out_ref[...] = jnp.concatenate([hn, ones, zeros], axis=1).astype(jnp.bfloat16)


def _prep_call(h, ln_s, ln_b, tile):
    n, c = h.shape
    grid = (n // tile,)
    return pl.pallas_call(
        _prep_body,
        grid=grid,
        in_specs=[
            pl.BlockSpec((tile, c), lambda i: (i, 0)),
            pl.BlockSpec((1, c), lambda i: (0, 0)),
            pl.BlockSpec((1, c), lambda i: (0, 0)),
        ],
        out_specs=pl.BlockSpec((tile, _W), lambda i: (i, 0)),
        out_shape=jax.ShapeDtypeStruct((n, _W), jnp.bfloat16),
    )(h, ln_s.reshape(1, c), ln_b.reshape(1, c))


def _block_body(r, c, hid, emit_next,
                h_ref, s_acc_ref,
                ln1s_ref, ln1b_ref, wg_ref, bg_ref, wrel_ref, brel_ref,
                wself_ref, bself_ref, wp_ref, bp_ref,
                ln2s_ref, ln2b_ref, w1_ref, b1_ref, w2_ref, b2_ref,
                nls_ref, nlb_ref,
                hout_ref, hn128_ref):
    h = h_ref[...]
    hn = _ln(h, ln1s_ref[...], ln1b_ref[...])
    logits = jnp.dot(hn, wg_ref[...], preferred_element_type=jnp.float32) + bg_ref[...]
    gmax = jnp.max(logits, axis=-1, keepdims=True)
    ge = jnp.exp(logits - gmax)
    gates = ge / jnp.sum(ge, axis=-1, keepdims=True)
    comb = _bfdot(hn, wself_ref[...])
    comb = comb + bself_ref[...]
    for rr in range(r):
        s_r = s_acc_ref[rr]
        cnt = s_r[:, c:c + 1]
        denom = jnp.maximum(cnt, 1.0)
        g_r = gates[:, rr:rr + 1]
        scaled = (g_r / denom) * s_r[:, :c]
        comb = comb + _bfdot(scaled, wrel_ref[rr])
        has = jnp.where(cnt > 0.0, 1.0, 0.0)
        comb = comb + (g_r * has) * brel_ref[rr:rr + 1, :]
    act = jax.nn.gelu(comb)
    out = _bfdot(act, wp_ref[...]) + bp_ref[...]
    h1 = h + out
    h2 = _ln(h1, ln2s_ref[...], ln2b_ref[...])
    ffn = jax.nn.gelu(_bfdot(h2, w1_ref[...]) + b1_ref[...])
    ffn = _bfdot(ffn, w2_ref[...]) + b2_ref[...]
    hf = h1 + ffn
    hout_ref[...] = hf
    if emit_next:
        t = hf.shape[0]
        hn_next = _ln(hf, nls_ref[...], nlb_ref[...])
        ones = jnp.ones((t, 1), jnp.float32)
        zeros = jnp.zeros((t, _W - c - 1), jnp.float32)
        hn128_ref[...] = jnp.concatenate([hn_next, ones, zeros], axis=1).astype(jnp.bfloat16)
    else:
        hn128_ref[...] = jnp.zeros_like(hn128_ref)


def _block_call(h, s_acc, blk, next_ln, tile):
    n, c = h.shape
    r = blk['b_rel'].shape[0]
    hid = blk['W1'].shape[1]
    emit_next = next_ln is not None
    nls = (next_ln[0] if emit_next else blk['ln1_s']).reshape(1, c)
    nlb = (next_ln[1] if emit_next else blk['ln1_b']).reshape(1, c)
    grid = (n // tile,)
    full = lambda a: pl.BlockSpec(a.shape, lambda i: (0,) * a.ndim)
    perm = []
    for g in range(3):
        perm += [32 * g + 2 * k for k in range(16)]
        perm += [32 * g + 2 * k + 1 for k in range(16)]
    wrel_p = blk['W_rel'][:, jnp.array(perm, jnp.int32), :]
    args = [
        blk['ln1_s'].reshape(1, c), blk['ln1_b'].reshape(1, c),
        blk['Wg'], blk['bg'].reshape(1, r),
        wrel_p, blk['b_rel'],
        blk['W_self'], blk['b_self'].reshape(1, c),
        blk['Wp'], blk['bp'].reshape(1, c),
        blk['ln2_s'].reshape(1, c), blk['ln2_b'].reshape(1, c),
        blk['W1'], blk['b1'].reshape(1, hid),
        blk['W2'], blk['b2'].reshape(1, c),
        nls, nlb,
    ]
    in_specs = [
        pl.BlockSpec((tile, c), lambda i: (i, 0)),
        pl.BlockSpec((r, tile, _W), lambda i: (0, i, 0)),
    ] + [full(a) for a in args]
    body = functools.partial(_block_body, r, c, hid, emit_next)
    return pl.pallas_call(
        body,
        grid=grid,
        in_specs=in_specs,
        out_specs=[
            pl.BlockSpec((tile, c), lambda i: (i, 0)),
            pl.BlockSpec((tile, _W), lambda i: (i, 0)),
        ],
        out_shape=[
            jax.ShapeDtypeStruct((n, c), jnp.float32),
            jax.ShapeDtypeStruct((n, _W), jnp.bfloat16),
        ],
    )(h, s_acc, *args)


def kernel(x, params, edge_index, edge_relation):
    b, l, c = x.shape
    n = b * l
    blocks = params['blocks']
    r = blocks[0]['b_rel'].shape[0]
    assert n == _N and r == _R and edge_index.shape[1] == _E
    src = edge_index[0]
    dst = edge_index[1]
    tile = 1024

    brs, brk, cntf = _bucket_call(src, dst, edge_relation)
    h = x.reshape(n, c)
    hn128 = _prep_call(h, blocks[0]['ln1_s'], blocks[0]['ln1_b'], tile)
    for bi, blk in enumerate(blocks):
        hn_tab = lax.bitcast_convert_type(hn128.reshape(n, _W // 2, 2), jnp.int32)
        (s_flat,) = _segsum_call(hn_tab, brs, brk, cntf)
        s_acc = s_flat.reshape(r, n, _W)
        nxt = None
        if bi + 1 < len(blocks):
            nxt = (blocks[bi + 1]['ln1_s'], blocks[bi + 1]['ln1_b'])
        h, hn128 = _block_call(h, s_acc, blk, nxt, tile)
    return h.reshape(b, l, c)
